# Initial kernel scaffold; baseline (speedup 1.0000x reference)
import functools

import jax
import jax.numpy as jnp
import numpy as np
from jax import lax
from jax.experimental import pallas as pl
from jax.experimental.pallas import tpu as pltpu
from jax.experimental.pallas import tpu_sc as plsc

_EPS = 1e-5
_SCALE = np.float32(1.0 / np.sqrt(1.0 + _EPS))
_R2_1 = np.float32(0.2 * 0.2)
_R2_2 = np.float32(0.4 * 0.4)


def _fps_body(m, x_ref, y_ref, z_ref, cx_ref, cy_ref, cz_ref):
    x = x_ref[...]
    y = y_ref[...]
    z = z_ref[...]
    B, N = x.shape
    iota_n = lax.broadcasted_iota(jnp.int32, (B, N), 1)
    iota_m = lax.broadcasted_iota(jnp.int32, (B, m), 1)

    def body(i, st):
        dists, last, cx, cy, cz = st
        onehot = iota_n == last
        xl = jnp.sum(jnp.where(onehot, x, 0.0), axis=1, keepdims=True)
        yl = jnp.sum(jnp.where(onehot, y, 0.0), axis=1, keepdims=True)
        zl = jnp.sum(jnp.where(onehot, z, 0.0), axis=1, keepdims=True)
        put = iota_m == (i - 1)
        cx = jnp.where(put, xl, cx)
        cy = jnp.where(put, yl, cy)
        cz = jnp.where(put, zl, cz)
        d = (x - xl) ** 2 + (y - yl) ** 2 + (z - zl) ** 2
        dists = jnp.minimum(dists, d)
        maxv = jnp.max(dists, axis=1, keepdims=True)
        nxt = jnp.min(jnp.where(dists == maxv, iota_n, N), axis=1, keepdims=True)
        return dists, nxt, cx, cy, cz

    dists0 = jnp.full((B, N), 1e10, dtype=jnp.float32)
    last0 = jnp.zeros((B, 1), dtype=jnp.int32)
    c0 = jnp.zeros((B, m), dtype=jnp.float32)
    _, _, cx, cy, cz = lax.fori_loop(1, m + 1, body, (dists0, last0, c0, c0, c0))
    cx_ref[...] = cx
    cy_ref[...] = cy
    cz_ref[...] = cz


def _fps(x, y, z, m):
    B, N = x.shape
    out = jax.ShapeDtypeStruct((B, m), jnp.float32)
    return pl.pallas_call(
        functools.partial(_fps_body, m),
        out_shape=(out, out, out),
    )(x, y, z)


def _cumsum_lanes(r):
    M, N = r.shape
    s = 1
    while s < N:
        shifted = jnp.concatenate(
            [jnp.zeros((M, s), dtype=r.dtype), r[:, : N - s]], axis=1
        )
        r = r + shifted
        s *= 2
    return r


def _first_k(mask, K, N, M, b):
    rank = _cumsum_lanes(mask.astype(jnp.int32))
    iota_k = lax.broadcasted_iota(jnp.int32, (M, K), 1)
    p = jnp.zeros((M, K), dtype=jnp.int32)
    for k in range(K):
        pk = jnp.sum((rank <= k).astype(jnp.int32), axis=1, keepdims=True)
        p = p + jnp.where(iota_k == k, pk, 0)
    p0 = jnp.broadcast_to(p[:, :1], (M, K))
    idx = jnp.where(p >= N, p0, p)
    return idx + b * N


def _bq1_body(N, M, K, C, r2, x_ref, y_ref, z_ref, xt_ref, yt_ref, zt_ref,
              cxt_ref, cyt_ref, czt_ref, wt_ref, b_ref,
              gidx_ref, wc_ref, a_ref):
    b = pl.program_id(0)
    xr = x_ref[...]
    yr = y_ref[...]
    zr = z_ref[...]
    cxs = cxt_ref[0]
    cys = cyt_ref[0]
    czs = czt_ref[0]

    d2 = (cxs - xr) ** 2 + (cys - yr) ** 2 + (czs - zr) ** 2
    gidx_ref[0] = _first_k(d2 <= r2, K, N, M, b)

    w0 = wt_ref[0:1, :]
    w1 = wt_ref[1:2, :]
    w2 = wt_ref[2:3, :]
    xt = xt_ref[0]
    yt = yt_ref[0]
    zt = zt_ref[0]
    a_ref[0] = xt * w0 + yt * w1 + zt * w2 + b_ref[...]
    wc_ref[0] = cxs * w0 + cys * w1 + czs * w2


def _bq1(x, y, z, cx, cy, cz, w, bias, M, K, r2):
    B, N = x.shape
    C = w.shape[0]
    wt = jnp.transpose(w)
    br = bias[None, :]
    xt = x[:, :, None]
    yt = y[:, :, None]
    zt = z[:, :, None]
    cxt = cx[:, :, None]
    cyt = cy[:, :, None]
    czt = cz[:, :, None]
    row = lambda i: (i, 0)
    row3 = lambda i: (i, 0, 0)
    rep = lambda i: (0, 0)
    return pl.pallas_call(
        functools.partial(_bq1_body, N, M, K, C, r2),
        grid=(B,),
        in_specs=[
            pl.BlockSpec((1, N), row), pl.BlockSpec((1, N), row), pl.BlockSpec((1, N), row),
            pl.BlockSpec((1, N, 1), row3), pl.BlockSpec((1, N, 1), row3), pl.BlockSpec((1, N, 1), row3),
            pl.BlockSpec((1, M, 1), row3), pl.BlockSpec((1, M, 1), row3), pl.BlockSpec((1, M, 1), row3),
            pl.BlockSpec((3, C), rep), pl.BlockSpec((1, C), rep),
        ],
        out_specs=[
            pl.BlockSpec((1, M, K), row3),
            pl.BlockSpec((1, M, C), row3),
            pl.BlockSpec((1, N, C), row3),
        ],
        out_shape=[
            jax.ShapeDtypeStruct((B, M, K), jnp.int32),
            jax.ShapeDtypeStruct((B, M, C), jnp.float32),
            jax.ShapeDtypeStruct((B, N, C), jnp.float32),
        ],
    )(x, y, z, xt, yt, zt, cxt, cyt, czt, wt, br)


def _bq2_body(N, M, K, C, r2, x_ref, y_ref, z_ref, xt_ref, yt_ref, zt_ref,
              cxt_ref, cyt_ref, czt_ref, f_ref, wxt_ref, wft_ref, b_ref,
              gidx_ref, wc_ref, a_ref):
    b = pl.program_id(0)
    xr = x_ref[...]
    yr = y_ref[...]
    zr = z_ref[...]
    cxs = cxt_ref[0]
    cys = cyt_ref[0]
    czs = czt_ref[0]

    d2 = (cxs - xr) ** 2 + (cys - yr) ** 2 + (czs - zr) ** 2
    gidx_ref[0] = _first_k(d2 <= r2, K, N, M, b)

    w0 = wxt_ref[0:1, :]
    w1 = wxt_ref[1:2, :]
    w2 = wxt_ref[2:3, :]
    xt = xt_ref[0]
    yt = yt_ref[0]
    zt = zt_ref[0]
    feat = f_ref[0]
    a = jnp.dot(feat, wft_ref[...], preferred_element_type=jnp.float32)
    a_ref[0] = a + xt * w0 + yt * w1 + zt * w2 + b_ref[...]
    wc_ref[0] = cxs * w0 + cys * w1 + czs * w2


def _bq2(x, y, z, cx, cy, cz, feat, w, bias, M, K, r2):
    B, N = x.shape
    C = w.shape[0]
    Cin = w.shape[1] - 3
    wxt = jnp.transpose(w[:, :3])
    wft = jnp.transpose(w[:, 3:])
    br = bias[None, :]
    xt = x[:, :, None]
    yt = y[:, :, None]
    zt = z[:, :, None]
    cxt = cx[:, :, None]
    cyt = cy[:, :, None]
    czt = cz[:, :, None]
    row = lambda i: (i, 0)
    row3 = lambda i: (i, 0, 0)
    rep = lambda i: (0, 0)
    return pl.pallas_call(
        functools.partial(_bq2_body, N, M, K, C, r2),
        grid=(B,),
        in_specs=[
            pl.BlockSpec((1, N), row), pl.BlockSpec((1, N), row), pl.BlockSpec((1, N), row),
            pl.BlockSpec((1, N, 1), row3), pl.BlockSpec((1, N, 1), row3), pl.BlockSpec((1, N, 1), row3),
            pl.BlockSpec((1, M, 1), row3), pl.BlockSpec((1, M, 1), row3), pl.BlockSpec((1, M, 1), row3),
            pl.BlockSpec((1, N, Cin), row3),
            pl.BlockSpec((3, C), rep), pl.BlockSpec((Cin, C), rep), pl.BlockSpec((1, C), rep),
        ],
        out_specs=[
            pl.BlockSpec((1, M, K), row3),
            pl.BlockSpec((1, M, C), row3),
            pl.BlockSpec((1, N, C), row3),
        ],
        out_shape=[
            jax.ShapeDtypeStruct((B, M, K), jnp.int32),
            jax.ShapeDtypeStruct((B, M, C), jnp.float32),
            jax.ShapeDtypeStruct((B, N, C), jnp.float32),
        ],
    )(x, y, z, xt, yt, zt, cxt, cyt, czt, feat, wxt, wft, br)


def _sc_gather(table, idx, ch):
    R, D = table.shape
    (B,) = idx.shape
    NW = 32
    b_per_w = B // NW
    nch = b_per_w // ch
    assert b_per_w % ch == 0 and B % NW == 0
    mesh = plsc.VectorSubcoreMesh(core_axis_name="c", subcore_axis_name="s")

    @functools.partial(
        pl.kernel,
        mesh=mesh,
        out_type=jax.ShapeDtypeStruct((B, D), jnp.float32),
        scratch_types=[
            pltpu.VMEM((ch,), jnp.int32),
            pltpu.VMEM((ch, D), jnp.float32),
            pltpu.SemaphoreType.DMA,
        ],
    )
    def k(table_hbm, idx_hbm, out_hbm, idx_v, rows_v, sem):
        wid = lax.axis_index("s") * 2 + lax.axis_index("c")
        base = wid * b_per_w

        def body(j, carry):
            off = base + j * ch
            pltpu.sync_copy(idx_hbm.at[pl.ds(off, ch)], idx_v)
            pltpu.async_copy(table_hbm.at[idx_v], rows_v, sem).wait()
            pltpu.sync_copy(rows_v, out_hbm.at[pl.ds(off, ch)])
            return carry

        lax.fori_loop(0, nch, body, 0)

    return k(table, idx)


def _mlp_body(MC, K, g_ref, wc_ref, w2_ref, b2_ref, w3_ref, b3_ref, o_ref):
    C1 = g_ref.shape[2]
    g = g_ref[0].reshape(MC, K, C1)
    wc = wc_ref[0]
    h = jax.nn.relu(_SCALE * (g - wc[:, None, :]))
    h = h.reshape(MC * K, C1)
    h = jax.nn.relu(_SCALE * (jnp.dot(h, w2_ref[...], preferred_element_type=jnp.float32) + b2_ref[...]))
    h = jnp.dot(h, w3_ref[...], preferred_element_type=jnp.float32) + b3_ref[...]
    h = jax.nn.relu(_SCALE * h)
    C3 = h.shape[1]
    o_ref[0] = jnp.max(h.reshape(MC, K, C3), axis=1)


def _group_mlp(g, wc, w2, b2, w3, b3, K, mc):
    B, MK, C1 = g.shape
    M = MK // K
    C3 = w3.shape[0]
    w2t = jnp.transpose(w2)
    w3t = jnp.transpose(w3)
    nchunk = M // mc
    rep = lambda b, i: (0, 0)
    return pl.pallas_call(
        functools.partial(_mlp_body, mc, K),
        grid=(B, nchunk),
        in_specs=[
            pl.BlockSpec((1, mc * K, C1), lambda b, i: (b, i, 0)),
            pl.BlockSpec((1, mc, C1), lambda b, i: (b, i, 0)),
            pl.BlockSpec((C1, C1), rep), pl.BlockSpec((1, C1), rep),
            pl.BlockSpec((C1, C3), rep), pl.BlockSpec((1, C3), rep),
        ],
        out_specs=pl.BlockSpec((1, mc, C3), lambda b, i: (b, i, 0)),
        out_shape=jax.ShapeDtypeStruct((B, M, C3), jnp.float32),
    )(g, wc, w2t, b2[None, :], w3t, b3[None, :])


def _head_body(f_ref, wl1_ref, bl1_ref, wl2_ref, bl2_ref, wl3_ref, bl3_ref,
               wg1_ref, bg1_ref, wg2_ref, bg2_ref, wc_ref, bc_ref, o_ref):
    v = f_ref[0]
    v = jax.nn.relu(_SCALE * (jnp.dot(v, wl1_ref[...], preferred_element_type=jnp.float32) + bl1_ref[...]))
    v = jax.nn.relu(_SCALE * (jnp.dot(v, wl2_ref[...], preferred_element_type=jnp.float32) + bl2_ref[...]))
    v = jax.nn.relu(_SCALE * (jnp.dot(v, wl3_ref[...], preferred_element_type=jnp.float32) + bl3_ref[...]))
    v = jnp.max(v, axis=0, keepdims=True)
    v = jax.nn.relu(_SCALE * (jnp.dot(v, wg1_ref[...], preferred_element_type=jnp.float32) + bg1_ref[...]))
    v = jax.nn.relu(_SCALE * (jnp.dot(v, wg2_ref[...], preferred_element_type=jnp.float32) + bg2_ref[...]))
    o_ref[0] = jnp.dot(v, wc_ref[...], preferred_element_type=jnp.float32) + bc_ref[...]


def _head(feat, local, glob, cls_w, cls_b):
    B, M, C = feat.shape
    (wl1, bl1), (wl2, bl2), (wl3, bl3) = local
    (wg1, bg1), (wg2, bg2) = glob
    NC = cls_w.shape[0]
    rep = lambda b: (0, 0)
    args = [
        (jnp.transpose(wl1), bl1[None, :]),
        (jnp.transpose(wl2), bl2[None, :]),
        (jnp.transpose(wl3), bl3[None, :]),
        (jnp.transpose(wg1), bg1[None, :]),
        (jnp.transpose(wg2), bg2[None, :]),
        (jnp.transpose(cls_w), cls_b[None, :]),
    ]
    in_specs = [pl.BlockSpec((1, M, C), lambda b: (b, 0, 0))]
    flat = []
    for (w, bb) in args:
        in_specs.append(pl.BlockSpec(w.shape, rep))
        in_specs.append(pl.BlockSpec(bb.shape, rep))
        flat.extend([w, bb])
    out = pl.pallas_call(
        _head_body,
        grid=(B,),
        in_specs=in_specs,
        out_specs=pl.BlockSpec((1, 1, NC), lambda b: (b, 0, 0)),
        out_shape=jax.ShapeDtypeStruct((B, 1, NC), jnp.float32),
    )(feat, *flat)
    return out.reshape(B, NC)


def kernel(points, params):
    B, _, N = points.shape
    x = points[:, 0, :]
    y = points[:, 1, :]
    z = points[:, 2, :]

    w1, b1 = params['sa1'][0]
    w2, b2 = params['sa1'][1]
    w3, b3 = params['sa1'][2]
    M1, K1 = 512, 32
    cx1, cy1, cz1 = _fps(x, y, z, M1)
    gidx1, wc1, a1 = _bq1(x, y, z, cx1, cy1, cz1, w1, b1, M1, K1, _R2_1)
    C1 = w1.shape[0]
    g1 = _sc_gather(a1.reshape(B * N, C1), gidx1.reshape(-1), ch=1024)
    feat1 = _group_mlp(g1.reshape(B, M1 * K1, C1), wc1, w2, b2, w3, b3, K1, mc=128)

    w1b, b1b = params['sa2'][0]
    w2b, b2b = params['sa2'][1]
    w3b, b3b = params['sa2'][2]
    M2, K2 = 128, 64
    cx2, cy2, cz2 = _fps(cx1, cy1, cz1, M2)
    gidx2, wc2, a2 = _bq2(cx1, cy1, cz1, cx2, cy2, cz2, feat1, w1b, b1b, M2, K2, _R2_2)
    C2 = w1b.shape[0]
    g2 = _sc_gather(a2.reshape(B * M1, C2), gidx2.reshape(-1), ch=512)
    feat2 = _group_mlp(g2.reshape(B, M2 * K2, C2), wc2, w2b, b2b, w3b, b3b, K2, mc=32)

    return _head(feat2, params['local'], params['global'], params['cls_w'], params['cls_b'])


# speedup vs baseline: 17.8055x; 17.8055x over previous
import functools

import jax
import jax.numpy as jnp
import numpy as np
from jax import lax
from jax.experimental import pallas as pl
from jax.experimental.pallas import tpu as pltpu
from jax.experimental.pallas import tpu_sc as plsc

_EPS = 1e-5
_SCALE = np.float32(1.0 / np.sqrt(1.0 + _EPS))
_R2_1 = np.float32(0.2 * 0.2)
_R2_2 = np.float32(0.4 * 0.4)


def _fps_body(m, x_ref, y_ref, z_ref, cx_ref, cy_ref, cz_ref):
    x = x_ref[...]
    y = y_ref[...]
    z = z_ref[...]
    B, N = x.shape
    iota_n = lax.broadcasted_iota(jnp.int32, (B, N), 1)
    iota_m = lax.broadcasted_iota(jnp.int32, (B, m), 1)

    def body(i, st):
        dists, last, cx, cy, cz = st
        onehot = iota_n == last
        xl = jnp.sum(jnp.where(onehot, x, 0.0), axis=1, keepdims=True)
        yl = jnp.sum(jnp.where(onehot, y, 0.0), axis=1, keepdims=True)
        zl = jnp.sum(jnp.where(onehot, z, 0.0), axis=1, keepdims=True)
        put = iota_m == (i - 1)
        cx = jnp.where(put, xl, cx)
        cy = jnp.where(put, yl, cy)
        cz = jnp.where(put, zl, cz)
        d = (x - xl) ** 2 + (y - yl) ** 2 + (z - zl) ** 2
        dists = jnp.minimum(dists, d)
        maxv = jnp.max(dists, axis=1, keepdims=True)
        nxt = jnp.min(jnp.where(dists == maxv, iota_n, N), axis=1, keepdims=True)
        return dists, nxt, cx, cy, cz

    dists0 = jnp.full((B, N), 1e10, dtype=jnp.float32)
    last0 = jnp.zeros((B, 1), dtype=jnp.int32)
    c0 = jnp.zeros((B, m), dtype=jnp.float32)
    _, _, cx, cy, cz = lax.fori_loop(1, m + 1, body, (dists0, last0, c0, c0, c0))
    cx_ref[...] = cx
    cy_ref[...] = cy
    cz_ref[...] = cz


def _fps(x, y, z, m):
    B, N = x.shape
    out = jax.ShapeDtypeStruct((B, m), jnp.float32)
    return pl.pallas_call(
        functools.partial(_fps_body, m),
        out_shape=(out, out, out),
    )(x, y, z)


def _cumsum_lanes(r):
    M, N = r.shape
    s = 1
    while s < N:
        shifted = jnp.concatenate(
            [jnp.zeros((M, s), dtype=r.dtype), r[:, : N - s]], axis=1
        )
        r = r + shifted
        s *= 2
    return r


def _first_k(mask, K, N, M, b):
    rank = _cumsum_lanes(mask.astype(jnp.int32))
    iota_k = lax.broadcasted_iota(jnp.int32, (M, K), 1)
    p = jnp.zeros((M, K), dtype=jnp.int32)
    for k in range(K):
        pk = jnp.sum((rank <= k).astype(jnp.int32), axis=1, keepdims=True)
        p = p + jnp.where(iota_k == k, pk, 0)
    p0 = jnp.broadcast_to(p[:, :1], (M, K))
    idx = jnp.where(p >= N, p0, p)
    return idx + b * N


def _bq1_body(N, M, K, C, r2, x_ref, y_ref, z_ref, xt_ref, yt_ref, zt_ref,
              cxt_ref, cyt_ref, czt_ref, wt_ref, b_ref,
              gidx_ref, wc_ref, a_ref):
    b = pl.program_id(0)
    xr = x_ref[0]
    yr = y_ref[0]
    zr = z_ref[0]
    cxs = cxt_ref[0]
    cys = cyt_ref[0]
    czs = czt_ref[0]

    d2 = (cxs - xr) ** 2 + (cys - yr) ** 2 + (czs - zr) ** 2
    gidx_ref[0] = _first_k(d2 <= r2, K, N, M, b)

    w0 = wt_ref[0:1, :]
    w1 = wt_ref[1:2, :]
    w2 = wt_ref[2:3, :]
    xt = xt_ref[0]
    yt = yt_ref[0]
    zt = zt_ref[0]
    a = xt * w0 + yt * w1 + zt * w2 + b_ref[...]
    a_ref[0] = jnp.concatenate([a, jnp.zeros((N, 128 - C), jnp.float32)], axis=1)
    wc_ref[0] = cxs * w0 + cys * w1 + czs * w2


def _bq1(x, y, z, cx, cy, cz, w, bias, M, K, r2):
    B, N = x.shape
    C = w.shape[0]
    wt = jnp.transpose(w)
    br = bias[None, :]
    xr = x[:, None, :]
    yr = y[:, None, :]
    zr = z[:, None, :]
    xt = x[:, :, None]
    yt = y[:, :, None]
    zt = z[:, :, None]
    cxt = cx[:, :, None]
    cyt = cy[:, :, None]
    czt = cz[:, :, None]
    row3 = lambda i: (i, 0, 0)
    rep = lambda i: (0, 0)
    return pl.pallas_call(
        functools.partial(_bq1_body, N, M, K, C, r2),
        grid=(B,),
        in_specs=[
            pl.BlockSpec((1, 1, N), row3), pl.BlockSpec((1, 1, N), row3), pl.BlockSpec((1, 1, N), row3),
            pl.BlockSpec((1, N, 1), row3), pl.BlockSpec((1, N, 1), row3), pl.BlockSpec((1, N, 1), row3),
            pl.BlockSpec((1, M, 1), row3), pl.BlockSpec((1, M, 1), row3), pl.BlockSpec((1, M, 1), row3),
            pl.BlockSpec((3, C), rep), pl.BlockSpec((1, C), rep),
        ],
        out_specs=[
            pl.BlockSpec((1, M, K), row3),
            pl.BlockSpec((1, M, C), row3),
            pl.BlockSpec((1, N, 128), row3),
        ],
        out_shape=[
            jax.ShapeDtypeStruct((B, M, K), jnp.int32),
            jax.ShapeDtypeStruct((B, M, C), jnp.float32),
            jax.ShapeDtypeStruct((B, N, 128), jnp.float32),
        ],
    )(xr, yr, zr, xt, yt, zt, cxt, cyt, czt, wt, br)


def _bq2_body(N, M, K, C, r2, x_ref, y_ref, z_ref, xt_ref, yt_ref, zt_ref,
              cxt_ref, cyt_ref, czt_ref, f_ref, wxt_ref, wft_ref, b_ref,
              gidx_ref, wc_ref, a_ref):
    b = pl.program_id(0)
    xr = x_ref[0]
    yr = y_ref[0]
    zr = z_ref[0]
    cxs = cxt_ref[0]
    cys = cyt_ref[0]
    czs = czt_ref[0]

    d2 = (cxs - xr) ** 2 + (cys - yr) ** 2 + (czs - zr) ** 2
    gidx_ref[0] = _first_k(d2 <= r2, K, N, M, b)

    w0 = wxt_ref[0:1, :]
    w1 = wxt_ref[1:2, :]
    w2 = wxt_ref[2:3, :]
    xt = xt_ref[0]
    yt = yt_ref[0]
    zt = zt_ref[0]
    feat = f_ref[0]
    a = jnp.dot(feat, wft_ref[...], preferred_element_type=jnp.float32)
    a_ref[0] = a + xt * w0 + yt * w1 + zt * w2 + b_ref[...]
    wc_ref[0] = cxs * w0 + cys * w1 + czs * w2


def _bq2(x, y, z, cx, cy, cz, feat, w, bias, M, K, r2):
    B, N = x.shape
    C = w.shape[0]
    Cin = w.shape[1] - 3
    wxt = jnp.transpose(w[:, :3])
    wft = jnp.transpose(w[:, 3:])
    br = bias[None, :]
    xr = x[:, None, :]
    yr = y[:, None, :]
    zr = z[:, None, :]
    xt = x[:, :, None]
    yt = y[:, :, None]
    zt = z[:, :, None]
    cxt = cx[:, :, None]
    cyt = cy[:, :, None]
    czt = cz[:, :, None]
    row3 = lambda i: (i, 0, 0)
    rep = lambda i: (0, 0)
    return pl.pallas_call(
        functools.partial(_bq2_body, N, M, K, C, r2),
        grid=(B,),
        in_specs=[
            pl.BlockSpec((1, 1, N), row3), pl.BlockSpec((1, 1, N), row3), pl.BlockSpec((1, 1, N), row3),
            pl.BlockSpec((1, N, 1), row3), pl.BlockSpec((1, N, 1), row3), pl.BlockSpec((1, N, 1), row3),
            pl.BlockSpec((1, M, 1), row3), pl.BlockSpec((1, M, 1), row3), pl.BlockSpec((1, M, 1), row3),
            pl.BlockSpec((1, N, Cin), row3),
            pl.BlockSpec((3, C), rep), pl.BlockSpec((Cin, C), rep), pl.BlockSpec((1, C), rep),
        ],
        out_specs=[
            pl.BlockSpec((1, M, K), row3),
            pl.BlockSpec((1, M, C), row3),
            pl.BlockSpec((1, N, C), row3),
        ],
        out_shape=[
            jax.ShapeDtypeStruct((B, M, K), jnp.int32),
            jax.ShapeDtypeStruct((B, M, C), jnp.float32),
            jax.ShapeDtypeStruct((B, N, C), jnp.float32),
        ],
    )(xr, yr, zr, xt, yt, zt, cxt, cyt, czt, feat, wxt, wft, br)


def _sc_gather(table, idx, ch):
    R, D = table.shape
    (B,) = idx.shape
    NW = 32
    b_per_w = B // NW
    nch = b_per_w // ch
    assert b_per_w % ch == 0 and B % NW == 0
    mesh = plsc.VectorSubcoreMesh(core_axis_name="c", subcore_axis_name="s")

    @functools.partial(
        pl.kernel,
        mesh=mesh,
        out_type=jax.ShapeDtypeStruct((B, D), jnp.float32),
        scratch_types=[
            pltpu.VMEM((ch,), jnp.int32),
            pltpu.VMEM((ch, D), jnp.float32),
            pltpu.SemaphoreType.DMA,
        ],
    )
    def k(table_hbm, idx_hbm, out_hbm, idx_v, rows_v, sem):
        wid = lax.axis_index("s") * 2 + lax.axis_index("c")
        base = wid * b_per_w

        def body(j, carry):
            off = base + j * ch
            pltpu.sync_copy(idx_hbm.at[pl.ds(off, ch)], idx_v)
            pltpu.async_copy(table_hbm.at[idx_v], rows_v, sem).wait()
            pltpu.sync_copy(rows_v, out_hbm.at[pl.ds(off, ch)])
            return carry

        lax.fori_loop(0, nch, body, 0)

    return k(table, idx)


def _mlp_body(MC, K, g_ref, wc_ref, w2_ref, b2_ref, w3_ref, b3_ref, o_ref):
    Cpad = g_ref.shape[2]
    C1 = wc_ref.shape[2]
    g = g_ref[0].reshape(MC, K, Cpad)
    if Cpad != C1:
        g = g[:, :, :C1]
    wc = wc_ref[0]
    h = jax.nn.relu(_SCALE * (g - wc[:, None, :]))
    h = h.reshape(MC * K, C1)
    h = jax.nn.relu(_SCALE * (jnp.dot(h, w2_ref[...], preferred_element_type=jnp.float32) + b2_ref[...]))
    h = jnp.dot(h, w3_ref[...], preferred_element_type=jnp.float32) + b3_ref[...]
    h = jax.nn.relu(_SCALE * h)
    C3 = h.shape[1]
    o_ref[0] = jnp.max(h.reshape(MC, K, C3), axis=1)


def _group_mlp(g, wc, w2, b2, w3, b3, K, mc):
    B, MK, Cpad = g.shape
    C1 = wc.shape[2]
    M = MK // K
    C3 = w3.shape[0]
    w2t = jnp.transpose(w2)
    w3t = jnp.transpose(w3)
    nchunk = M // mc
    rep = lambda b, i: (0, 0)
    return pl.pallas_call(
        functools.partial(_mlp_body, mc, K),
        grid=(B, nchunk),
        in_specs=[
            pl.BlockSpec((1, mc * K, Cpad), lambda b, i: (b, i, 0)),
            pl.BlockSpec((1, mc, C1), lambda b, i: (b, i, 0)),
            pl.BlockSpec((C1, C1), rep), pl.BlockSpec((1, C1), rep),
            pl.BlockSpec((C1, C3), rep), pl.BlockSpec((1, C3), rep),
        ],
        out_specs=pl.BlockSpec((1, mc, C3), lambda b, i: (b, i, 0)),
        out_shape=jax.ShapeDtypeStruct((B, M, C3), jnp.float32),
    )(g, wc, w2t, b2[None, :], w3t, b3[None, :])


def _head_body(f_ref, wl1_ref, bl1_ref, wl2_ref, bl2_ref, wl3_ref, bl3_ref,
               wg1_ref, bg1_ref, wg2_ref, bg2_ref, wc_ref, bc_ref, o_ref):
    v = f_ref[0]
    v = jax.nn.relu(_SCALE * (jnp.dot(v, wl1_ref[...], preferred_element_type=jnp.float32) + bl1_ref[...]))
    v = jax.nn.relu(_SCALE * (jnp.dot(v, wl2_ref[...], preferred_element_type=jnp.float32) + bl2_ref[...]))
    v = jax.nn.relu(_SCALE * (jnp.dot(v, wl3_ref[...], preferred_element_type=jnp.float32) + bl3_ref[...]))
    v = jnp.max(v, axis=0, keepdims=True)
    v = jax.nn.relu(_SCALE * (jnp.dot(v, wg1_ref[...], preferred_element_type=jnp.float32) + bg1_ref[...]))
    v = jax.nn.relu(_SCALE * (jnp.dot(v, wg2_ref[...], preferred_element_type=jnp.float32) + bg2_ref[...]))
    o_ref[0] = jnp.dot(v, wc_ref[...], preferred_element_type=jnp.float32) + bc_ref[...]


def _head(feat, local, glob, cls_w, cls_b):
    B, M, C = feat.shape
    (wl1, bl1), (wl2, bl2), (wl3, bl3) = local
    (wg1, bg1), (wg2, bg2) = glob
    NC = cls_w.shape[0]
    rep = lambda b: (0, 0)
    args = [
        (jnp.transpose(wl1), bl1[None, :]),
        (jnp.transpose(wl2), bl2[None, :]),
        (jnp.transpose(wl3), bl3[None, :]),
        (jnp.transpose(wg1), bg1[None, :]),
        (jnp.transpose(wg2), bg2[None, :]),
        (jnp.transpose(cls_w), cls_b[None, :]),
    ]
    in_specs = [pl.BlockSpec((1, M, C), lambda b: (b, 0, 0))]
    flat = []
    for (w, bb) in args:
        in_specs.append(pl.BlockSpec(w.shape, rep))
        in_specs.append(pl.BlockSpec(bb.shape, rep))
        flat.extend([w, bb])
    out = pl.pallas_call(
        _head_body,
        grid=(B,),
        in_specs=in_specs,
        out_specs=pl.BlockSpec((1, 1, NC), lambda b: (b, 0, 0)),
        out_shape=jax.ShapeDtypeStruct((B, 1, NC), jnp.float32),
    )(feat, *flat)
    return out.reshape(B, NC)


def kernel(points, params):
    B, _, N = points.shape
    x = points[:, 0, :]
    y = points[:, 1, :]
    z = points[:, 2, :]

    w1, b1 = params['sa1'][0]
    w2, b2 = params['sa1'][1]
    w3, b3 = params['sa1'][2]
    M1, K1 = 512, 32
    cx1, cy1, cz1 = _fps(x, y, z, M1)
    gidx1, wc1, a1 = _bq1(x, y, z, cx1, cy1, cz1, w1, b1, M1, K1, _R2_1)
    g1 = _sc_gather(a1.reshape(B * N, 128), gidx1.reshape(-1), ch=512)
    feat1 = _group_mlp(g1.reshape(B, M1 * K1, 128), wc1, w2, b2, w3, b3, K1, mc=128)

    w1b, b1b = params['sa2'][0]
    w2b, b2b = params['sa2'][1]
    w3b, b3b = params['sa2'][2]
    M2, K2 = 128, 64
    cx2, cy2, cz2 = _fps(cx1, cy1, cz1, M2)
    gidx2, wc2, a2 = _bq2(cx1, cy1, cz1, cx2, cy2, cz2, feat1, w1b, b1b, M2, K2, _R2_2)
    C2 = w1b.shape[0]
    g2 = _sc_gather(a2.reshape(B * M1, C2), gidx2.reshape(-1), ch=512)
    feat2 = _group_mlp(g2.reshape(B, M2 * K2, C2), wc2, w2b, b2b, w3b, b3b, K2, mc=32)

    return _head(feat2, params['local'], params['global'], params['cls_w'], params['cls_b'])
